# baseline (device time: 33530 ns/iter reference)
import jax
import jax.numpy as jnp
from jax import lax
from jax.experimental import pallas as pl
from jax.experimental.pallas import tpu as pltpu

M_PER = 1024
N_COLS = 512


def kernel(x, dest):
    d = dest.astype(jnp.int32)
    cum0 = jnp.cumsum((d == 0).astype(jnp.int32))
    aux = jnp.stack([d, cum0], axis=0)
    x16 = x

    def body(x_ref, aux_ref, out_ref, x_peer, aux_peer, send_sems, recv_sems):
        my_x = lax.axis_index("x")
        my_y = lax.axis_index("y")
        partner = (my_x, 1 - my_y)

        barrier_sem = pltpu.get_barrier_semaphore()
        pl.semaphore_signal(
            barrier_sem, inc=1,
            device_id=partner, device_id_type=pl.DeviceIdType.MESH,
        )
        pl.semaphore_wait(barrier_sem, 1)

        rdma_a = pltpu.make_async_remote_copy(
            src_ref=aux_ref, dst_ref=aux_peer,
            send_sem=send_sems.at[0], recv_sem=recv_sems.at[0],
            device_id=partner, device_id_type=pl.DeviceIdType.MESH,
        )
        rdma_a.start()
        rdma_x = pltpu.make_async_remote_copy(
            src_ref=x_ref, dst_ref=x_peer,
            send_sem=send_sems.at[1], recv_sem=recv_sems.at[1],
            device_id=partner, device_id_type=pl.DeviceIdType.MESH,
        )
        rdma_x.start()

        rdma_a.wait()

        iota_c = lax.broadcasted_iota(jnp.int32, (1, M_PER), 1)

        def decode(aux2):
            dest_r = aux2[0:1, :]
            c0 = aux2[1:2, :]
            cum_y = jnp.where(my_y == 0, c0, iota_c + 1 - c0)
            return dest_r == my_y, cum_y, jnp.max(cum_y)

        mask_m, cum_m, tot_m = decode(aux_ref[:, :])
        mask_p, cum_p, tot_p = decode(aux_peer[:, :])

        off_m = jnp.where(my_y == 0, 0, tot_p)
        off_p = jnp.where(my_y == 0, tot_m, 0)

        iota_p = lax.broadcasted_iota(jnp.int32, (M_PER, M_PER), 0)
        P_m = ((iota_p == (cum_m - 1 + off_m)) & mask_m).astype(jnp.float32)
        acc = jnp.dot(P_m, x_ref[:, :], preferred_element_type=jnp.float32)

        rdma_x.wait()
        P_p = ((iota_p == (cum_p - 1 + off_p)) & mask_p).astype(jnp.float32)
        acc = acc + jnp.dot(P_p, x_peer[:, :], preferred_element_type=jnp.float32)
        out_ref[:, :] = acc

    return pl.pallas_call(
        body,
        out_shape=jax.ShapeDtypeStruct((M_PER, N_COLS), jnp.float32),
        in_specs=[
            pl.BlockSpec(memory_space=pltpu.VMEM),
            pl.BlockSpec(memory_space=pltpu.VMEM),
        ],
        out_specs=pl.BlockSpec(memory_space=pltpu.VMEM),
        scratch_shapes=[
            pltpu.VMEM((M_PER, N_COLS), jnp.float32),
            pltpu.VMEM((2, M_PER), jnp.int32),
            pltpu.SemaphoreType.DMA((2,)),
            pltpu.SemaphoreType.DMA((2,)),
        ],
        compiler_params=pltpu.CompilerParams(collective_id=0),
    )(x16, aux)


# device time: 22352 ns/iter; 1.5001x vs baseline; 1.5001x over previous
import jax
import jax.numpy as jnp
from jax import lax
from jax.experimental import pallas as pl
from jax.experimental.pallas import tpu as pltpu

M_PER = 1024
N_COLS = 512


def kernel(x, dest):
    d = dest.astype(jnp.int32)
    cum0 = jnp.cumsum((d == 0).astype(jnp.int32))
    aux = jnp.stack([d, cum0], axis=0)
    x16 = x.astype(jnp.bfloat16)

    def body(x_ref, aux_ref, out_ref, x_peer, aux_peer, send_sems, recv_sems):
        my_x = lax.axis_index("x")
        my_y = lax.axis_index("y")
        partner = (my_x, 1 - my_y)

        barrier_sem = pltpu.get_barrier_semaphore()
        pl.semaphore_signal(
            barrier_sem, inc=1,
            device_id=partner, device_id_type=pl.DeviceIdType.MESH,
        )
        pl.semaphore_wait(barrier_sem, 1)

        rdma_a = pltpu.make_async_remote_copy(
            src_ref=aux_ref, dst_ref=aux_peer,
            send_sem=send_sems.at[0], recv_sem=recv_sems.at[0],
            device_id=partner, device_id_type=pl.DeviceIdType.MESH,
        )
        rdma_a.start()
        rdma_x = pltpu.make_async_remote_copy(
            src_ref=x_ref, dst_ref=x_peer,
            send_sem=send_sems.at[1], recv_sem=recv_sems.at[1],
            device_id=partner, device_id_type=pl.DeviceIdType.MESH,
        )
        rdma_x.start()

        rdma_a.wait()

        iota_c = lax.broadcasted_iota(jnp.int32, (1, M_PER), 1)

        def decode(aux2):
            dest_r = aux2[0:1, :]
            c0 = aux2[1:2, :]
            cum_y = jnp.where(my_y == 0, c0, iota_c + 1 - c0)
            return dest_r == my_y, cum_y, jnp.max(cum_y)

        mask_m, cum_m, tot_m = decode(aux_ref[:, :])
        mask_p, cum_p, tot_p = decode(aux_peer[:, :])

        off_m = jnp.where(my_y == 0, 0, tot_p)
        off_p = jnp.where(my_y == 0, tot_m, 0)

        iota_p = lax.broadcasted_iota(jnp.int32, (M_PER, M_PER), 0)
        P_m = ((iota_p == (cum_m - 1 + off_m)) & mask_m).astype(jnp.float32)
        xm = x_ref[:, :].astype(jnp.float32)
        acc = jnp.dot(P_m, xm, preferred_element_type=jnp.float32)

        rdma_x.wait()
        P_p = ((iota_p == (cum_p - 1 + off_p)) & mask_p).astype(jnp.float32)
        xp = x_peer[:, :].astype(jnp.float32)
        acc = acc + jnp.dot(P_p, xp, preferred_element_type=jnp.float32)
        out_ref[:, :] = acc

    return pl.pallas_call(
        body,
        out_shape=jax.ShapeDtypeStruct((M_PER, N_COLS), jnp.float32),
        in_specs=[
            pl.BlockSpec(memory_space=pltpu.VMEM),
            pl.BlockSpec(memory_space=pltpu.VMEM),
        ],
        out_specs=pl.BlockSpec(memory_space=pltpu.VMEM),
        scratch_shapes=[
            pltpu.VMEM((M_PER, N_COLS), jnp.bfloat16),
            pltpu.VMEM((2, M_PER), jnp.int32),
            pltpu.SemaphoreType.DMA((2,)),
            pltpu.SemaphoreType.DMA((2,)),
        ],
        compiler_params=pltpu.CompilerParams(collective_id=0),
    )(x16, aux)


# device time: 20492 ns/iter; 1.6362x vs baseline; 1.0908x over previous
import jax
import jax.numpy as jnp
from jax import lax
from jax.experimental import pallas as pl
from jax.experimental.pallas import tpu as pltpu

M_PER = 1024
N_COLS = 512
CHUNKS = 4
ROWS_C = M_PER // CHUNKS


def kernel(x, dest):
    d = dest.astype(jnp.int32)
    cum0 = jnp.cumsum((d == 0).astype(jnp.int32))
    aux = jnp.stack([d, cum0], axis=0)
    x16 = x.astype(jnp.bfloat16)

    def body(x_ref, aux_ref, out_ref, x_peer, aux_peer, send_sems, recv_sems):
        my_x = lax.axis_index("x")
        my_y = lax.axis_index("y")
        partner = (my_x, 1 - my_y)

        barrier_sem = pltpu.get_barrier_semaphore()
        pl.semaphore_signal(
            barrier_sem, inc=1,
            device_id=partner, device_id_type=pl.DeviceIdType.MESH,
        )
        pl.semaphore_wait(barrier_sem, 1)

        rdma_a = pltpu.make_async_remote_copy(
            src_ref=aux_ref, dst_ref=aux_peer,
            send_sem=send_sems.at[0], recv_sem=recv_sems.at[0],
            device_id=partner, device_id_type=pl.DeviceIdType.MESH,
        )
        rdma_a.start()
        rdma_x = []
        for k in range(CHUNKS):
            r = pltpu.make_async_remote_copy(
                src_ref=x_ref.at[pl.ds(k * ROWS_C, ROWS_C), :],
                dst_ref=x_peer.at[pl.ds(k * ROWS_C, ROWS_C), :],
                send_sem=send_sems.at[1 + k], recv_sem=recv_sems.at[1 + k],
                device_id=partner, device_id_type=pl.DeviceIdType.MESH,
            )
            r.start()
            rdma_x.append(r)

        rdma_a.wait()

        iota_c = lax.broadcasted_iota(jnp.int32, (1, M_PER), 1)

        def decode(aux2):
            dest_r = aux2[0:1, :]
            c0 = aux2[1:2, :]
            cum_y = jnp.where(my_y == 0, c0, iota_c + 1 - c0)
            return dest_r == my_y, cum_y, jnp.max(cum_y)

        mask_m, cum_m, tot_m = decode(aux_ref[:, :])
        mask_p, cum_p, tot_p = decode(aux_peer[:, :])

        off_m = jnp.where(my_y == 0, 0, tot_p)
        off_p = jnp.where(my_y == 0, tot_m, 0)

        iota_p = lax.broadcasted_iota(jnp.int32, (M_PER, M_PER), 0)
        P_m = ((iota_p == (cum_m - 1 + off_m)) & mask_m).astype(jnp.float32)
        P_p = ((iota_p == (cum_p - 1 + off_p)) & mask_p).astype(jnp.float32)
        xm = x_ref[:, :].astype(jnp.float32)
        acc = jnp.dot(P_m, xm, preferred_element_type=jnp.float32)

        for k in range(CHUNKS):
            rdma_x[k].wait()
            xp_k = x_peer[pl.ds(k * ROWS_C, ROWS_C), :].astype(jnp.float32)
            acc = acc + jnp.dot(
                P_p[:, k * ROWS_C:(k + 1) * ROWS_C], xp_k,
                preferred_element_type=jnp.float32,
            )
        out_ref[:, :] = acc

    return pl.pallas_call(
        body,
        out_shape=jax.ShapeDtypeStruct((M_PER, N_COLS), jnp.float32),
        in_specs=[
            pl.BlockSpec(memory_space=pltpu.VMEM),
            pl.BlockSpec(memory_space=pltpu.VMEM),
        ],
        out_specs=pl.BlockSpec(memory_space=pltpu.VMEM),
        scratch_shapes=[
            pltpu.VMEM((M_PER, N_COLS), jnp.bfloat16),
            pltpu.VMEM((2, M_PER), jnp.int32),
            pltpu.SemaphoreType.DMA((1 + CHUNKS,)),
            pltpu.SemaphoreType.DMA((1 + CHUNKS,)),
        ],
        compiler_params=pltpu.CompilerParams(collective_id=0),
    )(x16, aux)


# device time: 18937 ns/iter; 1.7706x vs baseline; 1.0821x over previous
import jax
import jax.numpy as jnp
from jax import lax
from jax.experimental import pallas as pl
from jax.experimental.pallas import tpu as pltpu

M_PER = 1024
N_COLS = 512
CHUNKS = 4
ROWS_C = M_PER // CHUNKS


def kernel(x, dest):
    dest2d = dest.astype(jnp.int32).reshape(1, M_PER)

    def body(x_ref, dest_ref, out_ref,
             x16_send, x_peer, dest_peer, send_sems, recv_sems):
        my_x = lax.axis_index("x")
        my_y = lax.axis_index("y")
        partner = (my_x, 1 - my_y)

        barrier_sem = pltpu.get_barrier_semaphore()
        pl.semaphore_signal(
            barrier_sem, inc=1,
            device_id=partner, device_id_type=pl.DeviceIdType.MESH,
        )
        pl.semaphore_wait(barrier_sem, 1)

        rdma_a = pltpu.make_async_remote_copy(
            src_ref=dest_ref, dst_ref=dest_peer,
            send_sem=send_sems.at[0], recv_sem=recv_sems.at[0],
            device_id=partner, device_id_type=pl.DeviceIdType.MESH,
        )
        rdma_a.start()
        rdma_x = []
        for k in range(CHUNKS):
            sl = pl.ds(k * ROWS_C, ROWS_C)
            x16_send[sl, :] = x_ref[sl, :].astype(jnp.bfloat16)
            r = pltpu.make_async_remote_copy(
                src_ref=x16_send.at[sl, :],
                dst_ref=x_peer.at[sl, :],
                send_sem=send_sems.at[1 + k], recv_sem=recv_sems.at[1 + k],
                device_id=partner, device_id_type=pl.DeviceIdType.MESH,
            )
            r.start()
            rdma_x.append(r)

        rdma_a.wait()

        iota_p = lax.broadcasted_iota(jnp.int32, (M_PER, M_PER), 0)
        iota_l = lax.broadcasted_iota(jnp.int32, (M_PER, M_PER), 1)
        tri = (iota_p <= iota_l).astype(jnp.float32)

        def cums(dest_row):
            mask = dest_row == my_y
            cum = jnp.dot(mask.astype(jnp.float32), tri,
                          preferred_element_type=jnp.float32)
            return mask, cum.astype(jnp.int32), jnp.max(cum).astype(jnp.int32)

        mask_m, cum_m, tot_m = cums(dest_ref[:, :])
        mask_p, cum_p, tot_p = cums(dest_peer[:, :])

        off_m = jnp.where(my_y == 0, 0, tot_p)
        off_p = jnp.where(my_y == 0, tot_m, 0)

        P_m = ((iota_p == (cum_m - 1 + off_m)) & mask_m).astype(jnp.float32)
        P_p = ((iota_p == (cum_p - 1 + off_p)) & mask_p).astype(jnp.float32)
        acc = jnp.dot(P_m, x_ref[:, :], preferred_element_type=jnp.float32)

        for k in range(CHUNKS):
            rdma_x[k].wait()
            xp_k = x_peer[pl.ds(k * ROWS_C, ROWS_C), :].astype(jnp.float32)
            acc = acc + jnp.dot(
                P_p[:, k * ROWS_C:(k + 1) * ROWS_C], xp_k,
                preferred_element_type=jnp.float32,
            )
        out_ref[:, :] = acc

    return pl.pallas_call(
        body,
        out_shape=jax.ShapeDtypeStruct((M_PER, N_COLS), jnp.float32),
        in_specs=[
            pl.BlockSpec(memory_space=pltpu.VMEM),
            pl.BlockSpec(memory_space=pltpu.VMEM),
        ],
        out_specs=pl.BlockSpec(memory_space=pltpu.VMEM),
        scratch_shapes=[
            pltpu.VMEM((M_PER, N_COLS), jnp.bfloat16),
            pltpu.VMEM((M_PER, N_COLS), jnp.bfloat16),
            pltpu.VMEM((1, M_PER), jnp.int32),
            pltpu.SemaphoreType.DMA((1 + CHUNKS,)),
            pltpu.SemaphoreType.DMA((1 + CHUNKS,)),
        ],
        compiler_params=pltpu.CompilerParams(collective_id=0),
    )(x, dest2d)
